# baseline (speedup 1.0000x reference)
import functools

import jax
import jax.numpy as jnp
from jax.experimental import pallas as pl
from jax.experimental.pallas import tpu as pltpu

N_P = 128
ND = 5
NDP = 8
HID = 64
FEATP = NDP * N_P

_HIGH = jax.lax.Precision.HIGHEST


def _dot(a, b):
    return jax.lax.dot(a, b, precision=_HIGH, preferred_element_type=jnp.float32)


_KB = 8
_RPP = 16


def _spatial_body(nodes0_ref, we1rT_ref, we1sT_ref, we1s_ref, pack64_ref,
                  packrow_ref, wn1n_ref, wn2T_ref, pack8_ref, dl_ref,
                  out_ref):
    we1rT = we1rT_ref[...]
    we1sT = we1sT_ref[...]
    we1s = we1s_ref[...]
    pack64 = pack64_ref[...]
    wd_col = pack64[:, 0:1]
    be1_col = pack64[:, 1:2]
    bn1_col = pack64[:, 2:3]
    wn1agg_col = pack64[:, 3:4]
    we2_row = packrow_ref[0:1, :]
    wn1n = wn1n_ref[...]
    wn2T = wn2T_ref[...]
    pack8 = pack8_ref[...]
    bn2_col = pack8[:, 0:1]
    be2 = pack8[0, 1]
    diag_dist = jnp.sqrt(jnp.float32(1e-12))
    wd3 = wd_col[None, :, :]

    def row_step(nodes, dl):
        ndT = nodes.T
        x_col = ndT[:, 0:1]
        y_col = ndT[:, 1:2]
        dx = x_col - nodes[0:1, :]
        dy = y_col - nodes[1:2, :]
        dist = jnp.sqrt(dx * dx + dy * dy + 1e-12)
        C = _dot(we1rT, nodes) + be1_col
        B = _dot(we1sT, nodes)
        BT = _dot(ndT, we1s)
        dist16 = dist.astype(jnp.bfloat16)
        C16 = C.astype(jnp.bfloat16)
        BT16 = BT.astype(jnp.bfloat16)
        wd16 = wd3.astype(jnp.bfloat16)
        S = jnp.zeros((HID, N_P), jnp.float32)
        for jb in range(0, N_P, _KB):
            H = (dist16[jb:jb + _KB, None, :] * wd16
                 + C16[None, :, :]
                 + BT16[jb:jb + _KB, :, None])
            t = jnp.tanh(H)
            n = _KB
            while n > 1:
                n //= 2
                t = t[:n] + t[n:2 * n]
            S = S + t[0].astype(jnp.float32)
        agg_pre = _dot(we2_row, S)
        Hd = wd_col * diag_dist + C + B
        e_diag = _dot(we2_row, jnp.tanh(Hd))
        agg = agg_pre - e_diag + 127.0 * be2
        Hn = wn1agg_col * agg + _dot(wn1n, nodes) + bn1_col
        new_nodes = _dot(wn2T, jnp.tanh(Hn)) + bn2_col
        return nodes + dl * new_nodes

    def step(s, carry):
        dl = dl_ref[s]
        return tuple(row_step(nodes, dl) for nodes in carry)

    init = tuple(nodes0_ref[r] for r in range(_RPP))
    final = jax.lax.fori_loop(0, 19, step, init)
    for r in range(_RPP):
        out_ref[r] = final[r]


def _temporal_body(hl_ref, x0_ref, wd1_ref, bd1_ref, wd2_ref, bd2_ref,
                   wf1_ref, bf1_ref, wf2_ref, bf2_ref, tt_ref, dtt_ref,
                   out_ref):
    bd1 = bd1_ref[0:1, :]
    bd2 = bd2_ref[0:1, :]
    bf1 = bf1_ref[0:1, :]
    bf2 = bf2_ref[0:1, :]
    Dh = jnp.tanh(_dot(hl_ref[...], wd1_ref[...]) + bd1)
    Dt = _dot(Dh, wd2_ref[...]) + bd2
    wf1 = wf1_ref[...]
    wf2 = wf2_ref[...]

    def step(s, x):
        f = _dot(jnp.tanh(_dot(x, wf1) + bf1), wf2) + bf2
        return x + dtt_ref[s] * (Dt + tt_ref[s] * f)

    out_ref[...] = jax.lax.fori_loop(0, 19, step, x0_ref[...])


def _pad_flat_rows(W):
    K = W.shape[1]
    Wr = W.reshape(N_P, ND, K).transpose(1, 0, 2).reshape(ND * N_P, K)
    return jnp.concatenate(
        [Wr, jnp.zeros(((NDP - ND) * N_P, K), jnp.float32)], axis=0)


def _pad_flat_cols(W):
    K = W.shape[0]
    Wr = W.reshape(K, N_P, ND).transpose(0, 2, 1).reshape(K, ND * N_P)
    return jnp.concatenate(
        [Wr, jnp.zeros((K, (NDP - ND) * N_P), jnp.float32)], axis=1)


def _pad_rows8(v):
    return jnp.concatenate([v[None, :], jnp.zeros((7, v.shape[0]), jnp.float32)], 0)


@functools.partial(jax.jit, static_argnums=())
def kernel(input_trajectory, dt, We1, be1, We2, be2, Wn1, bn1, Wn2, bn2,
           Wd1, bd1, Wd2, bd2, Wf1, bf1, Wf2, bf2):
    del dt
    B = input_trajectory.shape[1]

    nodes0 = jnp.squeeze(input_trajectory, 0)
    nodes0_t = jnp.transpose(nodes0, (0, 2, 1))
    nodes0_t = jnp.concatenate(
        [nodes0_t, jnp.zeros((B, NDP - ND, N_P), jnp.float32)], axis=1)

    zpad3 = jnp.zeros((HID, NDP - ND), jnp.float32)
    we1rT = jnp.concatenate([We1[1:6].T, zpad3], axis=1)
    we1sT = jnp.concatenate([We1[6:11].T, zpad3], axis=1)
    we1s = jnp.concatenate(
        [We1[6:11], jnp.zeros((NDP - ND, HID), jnp.float32)], axis=0)
    pack64 = jnp.stack(
        [We1[0], be1, bn1, Wn1[0], jnp.zeros_like(be1), jnp.zeros_like(be1),
         jnp.zeros_like(be1), jnp.zeros_like(be1)], axis=1)
    packrow = _pad_rows8(We2[:, 0])
    wn1n = jnp.concatenate([Wn1[1:6].T, zpad3], axis=1)
    wn2T = jnp.concatenate(
        [Wn2.T, jnp.zeros((NDP - ND, HID), jnp.float32)], axis=0)
    pack8 = jnp.zeros((8, 8), jnp.float32)
    pack8 = pack8.at[:ND, 0].set(bn2).at[0, 1].set(be2[0])

    L = jnp.linspace(0.0, 1.0, 20)
    dL = L[1:] - L[:-1]

    full = lambda shape: pl.BlockSpec(shape, lambda b: (0,) * len(shape))
    hl_t = pl.pallas_call(
        _spatial_body,
        grid=(B // _RPP,),
        in_specs=[
            pl.BlockSpec((_RPP, NDP, N_P), lambda b: (b, 0, 0)),
            full((HID, NDP)), full((HID, NDP)), full((NDP, HID)),
            full((HID, NDP)), full((NDP, HID)), full((HID, NDP)),
            full((NDP, HID)), full((NDP, NDP)),
            pl.BlockSpec(memory_space=pltpu.SMEM),
        ],
        out_specs=pl.BlockSpec((_RPP, NDP, N_P), lambda b: (b, 0, 0)),
        out_shape=jax.ShapeDtypeStruct((B, NDP, N_P), jnp.float32),
        compiler_params=pltpu.CompilerParams(
            dimension_semantics=("arbitrary",)),
    )(nodes0_t, we1rT, we1sT, we1s, pack64, packrow, wn1n, wn2T, pack8, dL)

    hl_flat = hl_t.reshape(B, FEATP)
    x0_flat = nodes0_t.reshape(B, FEATP)
    wd1p = _pad_flat_rows(Wd1)
    wd2p = _pad_flat_cols(Wd2)
    bd2p = _pad_rows8(_pad_flat_cols(bd2[None, :])[0])
    wf1p = _pad_flat_rows(Wf1)
    wf2p = _pad_flat_cols(Wf2)
    bf2p = _pad_rows8(_pad_flat_cols(bf2[None, :])[0])
    bd1p = _pad_rows8(bd1)
    bf1p = _pad_rows8(bf1)

    T = jnp.linspace(0.0, 1.0, 20)
    dT = T[1:] - T[:-1]
    tt = T[:-1]

    full0 = lambda shape: pl.BlockSpec(shape, lambda: (0,) * len(shape))
    xp = pl.pallas_call(
        _temporal_body,
        in_specs=[
            full0((B, FEATP)), full0((B, FEATP)),
            full0((FEATP, HID)), full0((NDP, HID)),
            full0((HID, FEATP)), full0((NDP, FEATP)),
            full0((FEATP, HID)), full0((NDP, HID)),
            full0((HID, FEATP)), full0((NDP, FEATP)),
            pl.BlockSpec(memory_space=pltpu.SMEM),
            pl.BlockSpec(memory_space=pltpu.SMEM),
        ],
        out_specs=full0((B, FEATP)),
        out_shape=jax.ShapeDtypeStruct((B, FEATP), jnp.float32),
    )(hl_flat, x0_flat, wd1p, bd1p, wd2p, bd2p, wf1p, bf1p, wf2p, bf2p,
      tt, dT)

    return xp.reshape(B, NDP, N_P).transpose(0, 2, 1)[:, :, :ND]


# speedup vs baseline: 1.0059x; 1.0059x over previous
import functools

import jax
import jax.numpy as jnp
from jax.experimental import pallas as pl
from jax.experimental.pallas import tpu as pltpu

N_P = 128
ND = 5
NDP = 8
HID = 64
FEATP = NDP * N_P

_HIGH = jax.lax.Precision.HIGHEST


def _dot(a, b):
    return jax.lax.dot(a, b, precision=_HIGH, preferred_element_type=jnp.float32)


_KB = 8
_RPP = 32


def _spatial_body(nodes0_ref, we1rT_ref, we1sT_ref, we1s_ref, pack64_ref,
                  packrow_ref, wn1n_ref, wn2T_ref, pack8_ref, dl_ref,
                  out_ref):
    we1rT = we1rT_ref[...]
    we1sT = we1sT_ref[...]
    we1s = we1s_ref[...]
    pack64 = pack64_ref[...]
    wd_col = pack64[:, 0:1]
    be1_col = pack64[:, 1:2]
    bn1_col = pack64[:, 2:3]
    wn1agg_col = pack64[:, 3:4]
    we2_row = packrow_ref[0:1, :]
    wn1n = wn1n_ref[...]
    wn2T = wn2T_ref[...]
    pack8 = pack8_ref[...]
    bn2_col = pack8[:, 0:1]
    be2 = pack8[0, 1]
    diag_dist = jnp.sqrt(jnp.float32(1e-12))
    wd3 = wd_col[None, :, :]

    def row_step(nodes, dl):
        ndT = nodes.T
        x_col = ndT[:, 0:1]
        y_col = ndT[:, 1:2]
        dx = x_col - nodes[0:1, :]
        dy = y_col - nodes[1:2, :]
        dist = jnp.sqrt(dx * dx + dy * dy + 1e-12)
        C = _dot(we1rT, nodes) + be1_col
        B = _dot(we1sT, nodes)
        BT = _dot(ndT, we1s)
        dist16 = dist.astype(jnp.bfloat16)
        C16 = C.astype(jnp.bfloat16)
        BT16 = BT.astype(jnp.bfloat16)
        wd16 = wd3.astype(jnp.bfloat16)
        S = jnp.zeros((HID, N_P), jnp.float32)
        for jb in range(0, N_P, _KB):
            H = (dist16[jb:jb + _KB, None, :] * wd16
                 + C16[None, :, :]
                 + BT16[jb:jb + _KB, :, None])
            t = jnp.tanh(H)
            n = _KB
            while n > 1:
                n //= 2
                t = t[:n] + t[n:2 * n]
            S = S + t[0].astype(jnp.float32)
        agg_pre = _dot(we2_row, S)
        Hd = wd_col * diag_dist + C + B
        e_diag = _dot(we2_row, jnp.tanh(Hd))
        agg = agg_pre - e_diag + 127.0 * be2
        Hn = wn1agg_col * agg + _dot(wn1n, nodes) + bn1_col
        new_nodes = _dot(wn2T, jnp.tanh(Hn)) + bn2_col
        return nodes + dl * new_nodes

    def step(s, carry):
        dl = dl_ref[s]
        return tuple(row_step(nodes, dl) for nodes in carry)

    init = tuple(nodes0_ref[r] for r in range(_RPP))
    final = jax.lax.fori_loop(0, 19, step, init)
    for r in range(_RPP):
        out_ref[r] = final[r]


def _temporal_body(hl_ref, x0_ref, wd1_ref, bd1_ref, wd2_ref, bd2_ref,
                   wf1_ref, bf1_ref, wf2_ref, bf2_ref, tt_ref, dtt_ref,
                   out_ref):
    bd1 = bd1_ref[0:1, :]
    bd2 = bd2_ref[0:1, :]
    bf1 = bf1_ref[0:1, :]
    bf2 = bf2_ref[0:1, :]
    Dh = jnp.tanh(_dot(hl_ref[...], wd1_ref[...]) + bd1)
    Dt = _dot(Dh, wd2_ref[...]) + bd2
    wf1 = wf1_ref[...]
    wf2 = wf2_ref[...]

    def step(s, x):
        f = _dot(jnp.tanh(_dot(x, wf1) + bf1), wf2) + bf2
        return x + dtt_ref[s] * (Dt + tt_ref[s] * f)

    out_ref[...] = jax.lax.fori_loop(0, 19, step, x0_ref[...])


def _pad_flat_rows(W):
    K = W.shape[1]
    Wr = W.reshape(N_P, ND, K).transpose(1, 0, 2).reshape(ND * N_P, K)
    return jnp.concatenate(
        [Wr, jnp.zeros(((NDP - ND) * N_P, K), jnp.float32)], axis=0)


def _pad_flat_cols(W):
    K = W.shape[0]
    Wr = W.reshape(K, N_P, ND).transpose(0, 2, 1).reshape(K, ND * N_P)
    return jnp.concatenate(
        [Wr, jnp.zeros((K, (NDP - ND) * N_P), jnp.float32)], axis=1)


def _pad_rows8(v):
    return jnp.concatenate([v[None, :], jnp.zeros((7, v.shape[0]), jnp.float32)], 0)


@functools.partial(jax.jit, static_argnums=())
def kernel(input_trajectory, dt, We1, be1, We2, be2, Wn1, bn1, Wn2, bn2,
           Wd1, bd1, Wd2, bd2, Wf1, bf1, Wf2, bf2):
    del dt
    B = input_trajectory.shape[1]

    nodes0 = jnp.squeeze(input_trajectory, 0)
    nodes0_t = jnp.transpose(nodes0, (0, 2, 1))
    nodes0_t = jnp.concatenate(
        [nodes0_t, jnp.zeros((B, NDP - ND, N_P), jnp.float32)], axis=1)

    zpad3 = jnp.zeros((HID, NDP - ND), jnp.float32)
    we1rT = jnp.concatenate([We1[1:6].T, zpad3], axis=1)
    we1sT = jnp.concatenate([We1[6:11].T, zpad3], axis=1)
    we1s = jnp.concatenate(
        [We1[6:11], jnp.zeros((NDP - ND, HID), jnp.float32)], axis=0)
    pack64 = jnp.stack(
        [We1[0], be1, bn1, Wn1[0], jnp.zeros_like(be1), jnp.zeros_like(be1),
         jnp.zeros_like(be1), jnp.zeros_like(be1)], axis=1)
    packrow = _pad_rows8(We2[:, 0])
    wn1n = jnp.concatenate([Wn1[1:6].T, zpad3], axis=1)
    wn2T = jnp.concatenate(
        [Wn2.T, jnp.zeros((NDP - ND, HID), jnp.float32)], axis=0)
    pack8 = jnp.zeros((8, 8), jnp.float32)
    pack8 = pack8.at[:ND, 0].set(bn2).at[0, 1].set(be2[0])

    L = jnp.linspace(0.0, 1.0, 20)
    dL = L[1:] - L[:-1]

    full = lambda shape: pl.BlockSpec(shape, lambda b: (0,) * len(shape))
    hl_t = pl.pallas_call(
        _spatial_body,
        grid=(B // _RPP,),
        in_specs=[
            pl.BlockSpec((_RPP, NDP, N_P), lambda b: (b, 0, 0)),
            full((HID, NDP)), full((HID, NDP)), full((NDP, HID)),
            full((HID, NDP)), full((NDP, HID)), full((HID, NDP)),
            full((NDP, HID)), full((NDP, NDP)),
            pl.BlockSpec(memory_space=pltpu.SMEM),
        ],
        out_specs=pl.BlockSpec((_RPP, NDP, N_P), lambda b: (b, 0, 0)),
        out_shape=jax.ShapeDtypeStruct((B, NDP, N_P), jnp.float32),
        compiler_params=pltpu.CompilerParams(
            dimension_semantics=("arbitrary",)),
    )(nodes0_t, we1rT, we1sT, we1s, pack64, packrow, wn1n, wn2T, pack8, dL)

    hl_flat = hl_t.reshape(B, FEATP)
    x0_flat = nodes0_t.reshape(B, FEATP)
    wd1p = _pad_flat_rows(Wd1)
    wd2p = _pad_flat_cols(Wd2)
    bd2p = _pad_rows8(_pad_flat_cols(bd2[None, :])[0])
    wf1p = _pad_flat_rows(Wf1)
    wf2p = _pad_flat_cols(Wf2)
    bf2p = _pad_rows8(_pad_flat_cols(bf2[None, :])[0])
    bd1p = _pad_rows8(bd1)
    bf1p = _pad_rows8(bf1)

    T = jnp.linspace(0.0, 1.0, 20)
    dT = T[1:] - T[:-1]
    tt = T[:-1]

    full0 = lambda shape: pl.BlockSpec(shape, lambda: (0,) * len(shape))
    xp = pl.pallas_call(
        _temporal_body,
        in_specs=[
            full0((B, FEATP)), full0((B, FEATP)),
            full0((FEATP, HID)), full0((NDP, HID)),
            full0((HID, FEATP)), full0((NDP, FEATP)),
            full0((FEATP, HID)), full0((NDP, HID)),
            full0((HID, FEATP)), full0((NDP, FEATP)),
            pl.BlockSpec(memory_space=pltpu.SMEM),
            pl.BlockSpec(memory_space=pltpu.SMEM),
        ],
        out_specs=full0((B, FEATP)),
        out_shape=jax.ShapeDtypeStruct((B, FEATP), jnp.float32),
    )(hl_flat, x0_flat, wd1p, bd1p, wd2p, bd2p, wf1p, bf1p, wf2p, bf2p,
      tt, dT)

    return xp.reshape(B, NDP, N_P).transpose(0, 2, 1)[:, :, :ND]
